# Initial kernel scaffold; baseline (speedup 1.0000x reference)
import functools

import jax
import jax.numpy as jnp
from jax import lax
from jax.experimental import pallas as pl
from jax.experimental.pallas import tpu as pltpu
from jax.experimental.pallas import tpu_sc as plsc

N_NODES = 100000
E_EDGES = 3200000
NC, NS = 2, 16
NW = NC * NS
EW = E_EDGES // NW
C = 2000
NCH = EW // C
P = 80
NP = 2 * C // P
G = C // 16

_HALF_PI = 1.5707963267948966


def _rsqrt_nr(x):
    i = plsc.bitcast(x, jnp.int32)
    i = jnp.int32(0x5F3759DF) - jnp.right_shift(i, 1)
    y = plsc.bitcast(i, jnp.float32)
    for _ in range(3):
        y = y * (1.5 - 0.5 * x * y * y)
    return y


def _atan01(r):
    r2 = r * r
    p = jnp.float32(-0.01172120)
    for c in (0.05265332, -0.11643287, 0.19354346, -0.33262347, 0.99997726):
        p = p * r2 + jnp.float32(c)
    return r * p


def _qmul(a, b):
    ax, ay, az, aw = a
    bx, by, bz, bw = b
    return (aw * bx + ax * bw + ay * bz - az * by,
            aw * by - ax * bz + ay * bw + az * bx,
            aw * bz + ax * by - ay * bx + az * bw,
            aw * bw - ax * bx - ay * by - az * bz)


def _cross(a, b):
    ax, ay, az = a
    bx, by, bz = b
    return (ay * bz - az * by, az * bx - ax * bz, ax * by - ay * bx)


def _qrot(q, v):
    qx, qy, qz, qw = q
    tx, ty, tz = _cross((qx, qy, qz), v)
    tx, ty, tz = 2.0 * tx, 2.0 * ty, 2.0 * tz
    cx, cy, cz = _cross((qx, qy, qz), (tx, ty, tz))
    return (v[0] + qw * tx + cx, v[1] + qw * ty + cy, v[2] + qw * tz + cz)


def _se3_log(t, q):
    tx, ty, tz = t
    qx, qy, qz, qw = q
    s = jnp.where(qw < 0.0, -1.0, 1.0)
    qx, qy, qz, qw = qx * s, qy * s, qz * s, qw * s
    n2 = qx * qx + qy * qy + qz * qz
    xe = n2 + 1e-24
    n = xe * _rsqrt_nr(xe)
    hi = jnp.maximum(n, qw)
    lo = jnp.minimum(n, qw)
    a = _atan01(lo / hi)
    half = jnp.where(n <= qw, a, jnp.float32(_HALF_PI) - a)
    theta = 2.0 * half
    small = n < 1e-6
    f = jnp.where(small, 2.0 / jnp.maximum(qw, 1e-6), theta / n)
    px, py, pz = f * qx, f * qy, f * qz
    h = n2 + qw * qw
    st = 2.0 * n * qw / h
    ct = (qw * qw - n2) / h
    one_m_ct = 1.0 - ct
    denom = 2.0 * (theta * theta) * one_m_ct
    safe = jnp.where(small, 1.0, denom)
    coef = jnp.where(small, jnp.float32(1.0 / 12.0),
                     (2.0 * one_m_ct - theta * st) / safe)
    cx, cy, cz = _cross((px, py, pz), (tx, ty, tz))
    dx, dy, dz = _cross((px, py, pz), (cx, cy, cz))
    return (tx - 0.5 * cx + coef * dx,
            ty - 0.5 * cy + coef * dy,
            tz - 0.5 * cz + coef * dz), (px, py, pz)


def _body(ed_hbm, pose_hbm, nodes_hbm, out_hbm, ed_v, pose_v, rows_v,
          out_v, sem):
    wid = lax.axis_index("s") * NC + lax.axis_index("c")
    ebase = wid * EW
    lane = lax.broadcasted_iota(jnp.int32, (16,), 0)
    cols7 = [jnp.full((16,), k, jnp.int32) for k in range(7)]

    @pl.loop(0, NCH)
    def _chunk(ci):
        base = ebase + ci * C
        pltpu.sync_copy(ed_hbm.at[pl.ds(base * 2 // P, NP)], ed_v)

        @pl.loop(0, NP)
        def _fire(p):
            pltpu.async_copy(nodes_hbm.at[ed_v.at[p]],
                             rows_v.at[pl.ds(p * P, P)], sem)

        pltpu.sync_copy(pose_hbm.at[pl.ds(base, C)], pose_v)
        pltpu.make_async_copy(nodes_hbm.at[pl.ds(0, 2 * C)], rows_v,
                              sem).wait()

        @pl.loop(0, G)
        def _group(g):
            r1 = 32 * g + 2 * lane
            r2 = r1 + 1
            rp = 16 * g + lane
            n1 = [plsc.load_gather(rows_v, [r1, cols7[k]]) for k in range(7)]
            n2 = [plsc.load_gather(rows_v, [r2, cols7[k]]) for k in range(7)]
            pp = [plsc.load_gather(pose_v, [rp, cols7[k]]) for k in range(7)]
            qa = (-n1[3], -n1[4], -n1[5], n1[6])
            qb = _qmul(qa, (n2[3], n2[4], n2[5], n2[6]))
            qp_i = (-pp[3], -pp[4], -pp[5], pp[6])
            qc = _qmul(qb, qp_i)
            tb = _qrot(qa, (n2[0] - n1[0], n2[1] - n1[1], n2[2] - n1[2]))
            rt = _qrot(qc, (pp[0], pp[1], pp[2]))
            tc = (tb[0] - rt[0], tb[1] - rt[1], tb[2] - rt[2])
            tau, phi = _se3_log(tc, qc)
            outs = (tau[0], tau[1], tau[2], phi[0], phi[1], phi[2])
            for k in range(6):
                plsc.store_scatter(out_v, [rp, jnp.full((16,), k, jnp.int32)],
                                   outs[k])

        pltpu.sync_copy(out_v, out_hbm.at[pl.ds(base, C)])


@jax.jit
def _pose_graph_sc(edges_r, poses, nodes8):
    run = functools.partial(
        pl.kernel,
        out_type=jax.ShapeDtypeStruct((E_EDGES, 6), jnp.float32),
        mesh=plsc.VectorSubcoreMesh(core_axis_name="c", subcore_axis_name="s",
                                    num_cores=NC, num_subcores=NS),
        scratch_types=[
            pltpu.VMEM((NP, P), jnp.int32),
            pltpu.VMEM((C, 7), jnp.float32),
            pltpu.VMEM((2 * C, 8), jnp.float32),
            pltpu.VMEM((C, 6), jnp.float32),
            pltpu.SemaphoreType.DMA,
        ],
    )(_body)
    return run(edges_r, poses, nodes8)


def kernel(edges, poses, nodes):
    nodes8 = jnp.concatenate(
        [nodes, jnp.zeros((nodes.shape[0], 1), nodes.dtype)], axis=1)
    edges_r = edges.astype(jnp.int32).reshape(2 * E_EDGES // P, P)
    return _pose_graph_sc(edges_r, poses, nodes8)


# speedup vs baseline: 3.2057x; 3.2057x over previous
import functools

import jax
import jax.numpy as jnp
from jax import lax
from jax.experimental import pallas as pl
from jax.experimental.pallas import tpu as pltpu
from jax.experimental.pallas import tpu_sc as plsc

N_NODES = 100000
E_EDGES = 3200000
NC, NS = 2, 16
NW = NC * NS
EW = E_EDGES // NW
C = 2000
NCH = EW // C
P = 100
NP = 2 * C // P
G = C // 16

_HALF_PI = 1.5707963267948966


def _rsqrt_nr(x):
    i = plsc.bitcast(x, jnp.int32)
    i = jnp.int32(0x5F3759DF) - jnp.right_shift(i, 1)
    y = plsc.bitcast(i, jnp.float32)
    for _ in range(3):
        y = y * (1.5 - 0.5 * x * y * y)
    return y


def _atan01(r):
    r2 = r * r
    p = jnp.float32(-0.01172120)
    for c in (0.05265332, -0.11643287, 0.19354346, -0.33262347, 0.99997726):
        p = p * r2 + jnp.float32(c)
    return r * p


def _qmul(a, b):
    ax, ay, az, aw = a
    bx, by, bz, bw = b
    return (aw * bx + ax * bw + ay * bz - az * by,
            aw * by - ax * bz + ay * bw + az * bx,
            aw * bz + ax * by - ay * bx + az * bw,
            aw * bw - ax * bx - ay * by - az * bz)


def _cross(a, b):
    ax, ay, az = a
    bx, by, bz = b
    return (ay * bz - az * by, az * bx - ax * bz, ax * by - ay * bx)


def _qrot(q, v):
    qx, qy, qz, qw = q
    tx, ty, tz = _cross((qx, qy, qz), v)
    tx, ty, tz = 2.0 * tx, 2.0 * ty, 2.0 * tz
    cx, cy, cz = _cross((qx, qy, qz), (tx, ty, tz))
    return (v[0] + qw * tx + cx, v[1] + qw * ty + cy, v[2] + qw * tz + cz)


def _se3_log(t, q):
    tx, ty, tz = t
    qx, qy, qz, qw = q
    s = jnp.where(qw < 0.0, -1.0, 1.0)
    qx, qy, qz, qw = qx * s, qy * s, qz * s, qw * s
    n2 = qx * qx + qy * qy + qz * qz
    xe = n2 + 1e-24
    n = xe * _rsqrt_nr(xe)
    hi = jnp.maximum(n, qw)
    lo = jnp.minimum(n, qw)
    a = _atan01(lo / hi)
    half = jnp.where(n <= qw, a, jnp.float32(_HALF_PI) - a)
    theta = 2.0 * half
    small = n < 1e-6
    f = jnp.where(small, 2.0 / jnp.maximum(qw, 1e-6), theta / n)
    px, py, pz = f * qx, f * qy, f * qz
    h = n2 + qw * qw
    st = 2.0 * n * qw / h
    ct = (qw * qw - n2) / h
    one_m_ct = 1.0 - ct
    denom = 2.0 * (theta * theta) * one_m_ct
    safe = jnp.where(small, 1.0, denom)
    coef = jnp.where(small, jnp.float32(1.0 / 12.0),
                     (2.0 * one_m_ct - theta * st) / safe)
    cx, cy, cz = _cross((px, py, pz), (tx, ty, tz))
    dx, dy, dz = _cross((px, py, pz), (cx, cy, cz))
    return (tx - 0.5 * cx + coef * dx,
            ty - 0.5 * cy + coef * dy,
            tz - 0.5 * cz + coef * dz), (px, py, pz)


def _body(ed_hbm, pose_hbm, nodes_hbm, out_hbm, ed_v, pose_v, rows_v,
          out_v, sem):
    wid = lax.axis_index("s") * NC + lax.axis_index("c")
    ebase = wid * EW
    lane = lax.broadcasted_iota(jnp.int32, (16,), 0)
    cols7 = [jnp.full((16,), k, jnp.int32) for k in range(7)]

    @pl.loop(0, NCH)
    def _chunk(ci):
        base = pl.multiple_of(ebase + ci * C, 8)
        edrow = pl.multiple_of(wid * (2 * EW // P) + ci * NP, 8)
        pltpu.sync_copy(ed_hbm.at[pl.ds(edrow, NP)], ed_v)

        @pl.loop(0, NP)
        def _fire(p):
            pltpu.async_copy(nodes_hbm.at[ed_v.at[p]],
                             rows_v.at[pl.ds(p * P, P)], sem)

        pltpu.sync_copy(pose_hbm.at[pl.ds(base, C)], pose_v)
        pltpu.make_async_copy(nodes_hbm.at[pl.ds(0, 2 * C)], rows_v,
                              sem).wait()

        @pl.loop(0, G)
        def _group(g):
            r1 = 32 * g + 2 * lane
            r2 = r1 + 1
            rp = 16 * g + lane
            n1 = [plsc.load_gather(rows_v, [r1, cols7[k]]) for k in range(7)]
            n2 = [plsc.load_gather(rows_v, [r2, cols7[k]]) for k in range(7)]
            pp = [plsc.load_gather(pose_v, [rp, cols7[k]]) for k in range(7)]
            qa = (-n1[3], -n1[4], -n1[5], n1[6])
            qb = _qmul(qa, (n2[3], n2[4], n2[5], n2[6]))
            qp_i = (-pp[3], -pp[4], -pp[5], pp[6])
            qc = _qmul(qb, qp_i)
            tb = _qrot(qa, (n2[0] - n1[0], n2[1] - n1[1], n2[2] - n1[2]))
            rt = _qrot(qc, (pp[0], pp[1], pp[2]))
            tc = (tb[0] - rt[0], tb[1] - rt[1], tb[2] - rt[2])
            tau, phi = _se3_log(tc, qc)
            outs = (tau[0], tau[1], tau[2], phi[0], phi[1], phi[2])
            for k in range(6):
                plsc.store_scatter(out_v, [rp, jnp.full((16,), k, jnp.int32)],
                                   outs[k])

        pltpu.sync_copy(out_v, out_hbm.at[pl.ds(base, C)])


@jax.jit
def _pose_graph_sc(edges_r, poses, nodes8):
    run = functools.partial(
        pl.kernel,
        out_type=jax.ShapeDtypeStruct((E_EDGES, 6), jnp.float32),
        mesh=plsc.VectorSubcoreMesh(core_axis_name="c", subcore_axis_name="s",
                                    num_cores=NC, num_subcores=NS),
        compiler_params=pltpu.CompilerParams(use_tc_tiling_on_sc=False,
                                             needs_layout_passes=False),
        scratch_types=[
            pltpu.VMEM((NP, P), jnp.int32),
            pltpu.VMEM((C, 7), jnp.float32),
            pltpu.VMEM((2 * C, 8), jnp.float32),
            pltpu.VMEM((C, 6), jnp.float32),
            pltpu.SemaphoreType.DMA,
        ],
    )(_body)
    return run(edges_r, poses, nodes8)


def kernel(edges, poses, nodes):
    nodes8 = jnp.concatenate(
        [nodes, jnp.zeros((nodes.shape[0], 1), nodes.dtype)], axis=1)
    edges_r = edges.astype(jnp.int32).reshape(2 * E_EDGES // P, P)
    return _pose_graph_sc(edges_r, poses, nodes8)


# speedup vs baseline: 16.0759x; 5.0149x over previous
import functools

import jax
import jax.numpy as jnp
from jax import lax
from jax.experimental import pallas as pl
from jax.experimental.pallas import tpu as pltpu
from jax.experimental.pallas import tpu_sc as plsc

N_NODES = 100000
E_EDGES = 3200000
NC, NS = 2, 16
NW = NC * NS
EW = E_EDGES // NW
C = 2000
NCH = EW // C
P = 80
NP = C // P
G = C // 16

_HALF_PI = 1.5707963267948966


def _rsqrt_nr(x):
    i = plsc.bitcast(x, jnp.int32)
    i = jnp.int32(0x5F3759DF) - jnp.right_shift(i, 1)
    y = plsc.bitcast(i, jnp.float32)
    for _ in range(3):
        y = y * (1.5 - 0.5 * x * y * y)
    return y


def _atan01(r):
    r2 = r * r
    p = jnp.float32(-0.01172120)
    for c in (0.05265332, -0.11643287, 0.19354346, -0.33262347, 0.99997726):
        p = p * r2 + jnp.float32(c)
    return r * p


def _qmul(a, b):
    ax, ay, az, aw = a
    bx, by, bz, bw = b
    return (aw * bx + ax * bw + ay * bz - az * by,
            aw * by - ax * bz + ay * bw + az * bx,
            aw * bz + ax * by - ay * bx + az * bw,
            aw * bw - ax * bx - ay * by - az * bz)


def _cross(a, b):
    ax, ay, az = a
    bx, by, bz = b
    return (ay * bz - az * by, az * bx - ax * bz, ax * by - ay * bx)


def _qrot(q, v):
    qx, qy, qz, qw = q
    tx, ty, tz = _cross((qx, qy, qz), v)
    tx, ty, tz = 2.0 * tx, 2.0 * ty, 2.0 * tz
    cx, cy, cz = _cross((qx, qy, qz), (tx, ty, tz))
    return (v[0] + qw * tx + cx, v[1] + qw * ty + cy, v[2] + qw * tz + cz)


def _se3_log(t, q):
    tx, ty, tz = t
    qx, qy, qz, qw = q
    s = jnp.where(qw < 0.0, -1.0, 1.0)
    qx, qy, qz, qw = qx * s, qy * s, qz * s, qw * s
    n2 = qx * qx + qy * qy + qz * qz
    xe = n2 + 1e-24
    n = xe * _rsqrt_nr(xe)
    hi = jnp.maximum(n, qw)
    lo = jnp.minimum(n, qw)
    a = _atan01(lo / hi)
    half = jnp.where(n <= qw, a, jnp.float32(_HALF_PI) - a)
    theta = 2.0 * half
    small = n < 1e-6
    f = jnp.where(small, 2.0 / jnp.maximum(qw, 1e-6), theta / n)
    px, py, pz = f * qx, f * qy, f * qz
    h = n2 + qw * qw
    st = 2.0 * n * qw / h
    ct = (qw * qw - n2) / h
    one_m_ct = 1.0 - ct
    denom = 2.0 * (theta * theta) * one_m_ct
    safe = jnp.where(small, 1.0, denom)
    coef = jnp.where(small, jnp.float32(1.0 / 12.0),
                     (2.0 * one_m_ct - theta * st) / safe)
    cx, cy, cz = _cross((px, py, pz), (tx, ty, tz))
    dx, dy, dz = _cross((px, py, pz), (cx, cy, cz))
    return (tx - 0.5 * cx + coef * dx,
            ty - 0.5 * cy + coef * dy,
            tz - 0.5 * cz + coef * dz), (px, py, pz)


def _body(e1_hbm, e2_hbm, p0, p1, p2, p3, p4, p5, p6, nodes_hbm,
          o0, o1, o2, o3, o4, o5,
          e1_v, e2_v, rows1_v, rows2_v,
          pv0, pv1, pv2, pv3, pv4, pv5, pv6,
          ov0, ov1, ov2, ov3, ov4, ov5, sem):
    p_hbm = (p0, p1, p2, p3, p4, p5, p6)
    o_hbm = (o0, o1, o2, o3, o4, o5)
    p_v = (pv0, pv1, pv2, pv3, pv4, pv5, pv6)
    o_v = (ov0, ov1, ov2, ov3, ov4, ov5)
    wid = lax.axis_index("s") * NC + lax.axis_index("c")
    ebase = wid * EW
    lane = lax.broadcasted_iota(jnp.int32, (16,), 0)
    cols7 = [jnp.full((16,), k, jnp.int32) for k in range(7)]

    @pl.loop(0, NCH)
    def _chunk(ci):
        base = pl.multiple_of(ebase + ci * C, C)
        pltpu.sync_copy(e1_hbm.at[pl.ds(base, C)], e1_v)
        pltpu.sync_copy(e2_hbm.at[pl.ds(base, C)], e2_v)

        @pl.loop(0, NP)
        def _fire(p):
            o = pl.multiple_of(p * P, P)
            pltpu.async_copy(nodes_hbm.at[e1_v.at[pl.ds(o, P)]],
                             rows1_v.at[pl.ds(o, P)], sem)
            pltpu.async_copy(nodes_hbm.at[e2_v.at[pl.ds(o, P)]],
                             rows2_v.at[pl.ds(o, P)], sem)

        for k in range(7):
            pltpu.sync_copy(p_hbm[k].at[pl.ds(base, C)], p_v[k])
        pltpu.make_async_copy(nodes_hbm.at[pl.ds(0, C)], rows1_v, sem).wait()
        pltpu.make_async_copy(nodes_hbm.at[pl.ds(0, C)], rows2_v, sem).wait()

        @pl.loop(0, G)
        def _group(g):
            r = 16 * g + 2 * lane
            rr = 16 * g + lane
            sl = pl.ds(pl.multiple_of(16 * g, 16), 16)
            n1 = [plsc.load_gather(rows1_v, [rr, cols7[k]]) for k in range(7)]
            n2 = [plsc.load_gather(rows2_v, [rr, cols7[k]]) for k in range(7)]
            pp = [p_v[k][sl] for k in range(7)]
            qa = (-n1[3], -n1[4], -n1[5], n1[6])
            qb = _qmul(qa, (n2[3], n2[4], n2[5], n2[6]))
            qp_i = (-pp[3], -pp[4], -pp[5], pp[6])
            qc = _qmul(qb, qp_i)
            tb = _qrot(qa, (n2[0] - n1[0], n2[1] - n1[1], n2[2] - n1[2]))
            rt = _qrot(qc, (pp[0], pp[1], pp[2]))
            tc = (tb[0] - rt[0], tb[1] - rt[1], tb[2] - rt[2])
            tau, phi = _se3_log(tc, qc)
            outs = (tau[0], tau[1], tau[2], phi[0], phi[1], phi[2])
            for k in range(6):
                o_v[k][sl] = outs[k]

        for k in range(6):
            pltpu.sync_copy(o_v[k], o_hbm[k].at[pl.ds(base, C)])


@jax.jit
def _pose_graph_sc(e1, e2, pcols, nodes8):
    run = functools.partial(
        pl.kernel,
        out_type=tuple(jax.ShapeDtypeStruct((E_EDGES,), jnp.float32)
                       for _ in range(6)),
        mesh=plsc.VectorSubcoreMesh(core_axis_name="c", subcore_axis_name="s",
                                    num_cores=NC, num_subcores=NS),
        compiler_params=pltpu.CompilerParams(use_tc_tiling_on_sc=False,
                                             needs_layout_passes=False),
        scratch_types=(
            [pltpu.VMEM((C,), jnp.int32)] * 2
            + [pltpu.VMEM((C, 8), jnp.float32)] * 2
            + [pltpu.VMEM((C,), jnp.float32)] * 13
            + [pltpu.SemaphoreType.DMA]
        ),
    )(_body)
    return run(e1, e2, *pcols, nodes8)


def kernel(edges, poses, nodes):
    nodes8 = jnp.concatenate(
        [nodes, jnp.zeros((nodes.shape[0], 1), nodes.dtype)], axis=1)
    e1 = edges[:, 0].astype(jnp.int32)
    e2 = edges[:, 1].astype(jnp.int32)
    pcols = [poses[:, k] for k in range(7)]
    outs = _pose_graph_sc(e1, e2, pcols, nodes8)
    return jnp.stack(outs, axis=1)
